# baseline (speedup 1.0000x reference)
import dataclasses
import functools

import jax
import jax.numpy as jnp
from jax import lax
from jax.experimental import pallas as pl
from jax.experimental.pallas import tpu as pltpu
from jax.experimental.pallas import tpu_sc as plsc

E = 8
TOPX = 2
D = 1024
N = 2048
NTOP = N * TOPX

BT = 128
S = NTOP + E * BT
T_PAD = S // BT

BTOK = 512
EPAD = 128

SC_NC = 2
SC_NS = 16
SC_NW = SC_NC * SC_NS

NEG = -1e30


def _sc_compiler_params():
    cp = pltpu.CompilerParams()
    if "needs_layout_passes" in pltpu.CompilerParams.__dataclass_fields__:
        cp = dataclasses.replace(cp, needs_layout_passes=False)
    return cp


def _router_body(x_ref, wg_ref, bg_ref, i1_ref, i2_ref, q1_ref, q2_ref):
    logits = jnp.dot(x_ref[...], wg_ref[...],
                     preferred_element_type=jnp.float32)
    logits = logits + bg_ref[...]
    ids = lax.broadcasted_iota(jnp.int32, logits.shape, 1)
    valid = ids < E
    logits = jnp.where(valid, logits, NEG)
    m1 = jnp.max(logits, axis=1, keepdims=True)
    i1 = jnp.min(jnp.where(logits == m1, ids, E), axis=1, keepdims=True)
    rest = jnp.where((ids != i1) & valid, logits, NEG)
    m2 = jnp.max(rest, axis=1, keepdims=True)
    i2 = jnp.min(jnp.where(rest == m2, ids, E), axis=1, keepdims=True)
    ed = jnp.exp(m2 - m1)
    p1 = 1.0 / (1.0 + ed)
    p2 = ed / (1.0 + ed)
    e21 = jnp.exp(p2 - p1)
    i1_ref[...] = i1
    i2_ref[...] = i2
    q1_ref[...] = 1.0 / (1.0 + e21)
    q2_ref[...] = e21 / (1.0 + e21)


def _router(x, wg_pad, bg_pad, *, interpret=False):
    grid = (N // BTOK,)
    out1 = jax.ShapeDtypeStruct((N, 1), jnp.int32)
    outf = jax.ShapeDtypeStruct((N, 1), jnp.float32)
    return pl.pallas_call(
        _router_body,
        grid=grid,
        in_specs=[
            pl.BlockSpec((BTOK, D), lambda t: (t, 0)),
            pl.BlockSpec((D, EPAD), lambda t: (0, 0)),
            pl.BlockSpec((1, EPAD), lambda t: (0, 0)),
        ],
        out_specs=[
            pl.BlockSpec((BTOK, 1), lambda t: (t, 0)),
            pl.BlockSpec((BTOK, 1), lambda t: (t, 0)),
            pl.BlockSpec((BTOK, 1), lambda t: (t, 0)),
            pl.BlockSpec((BTOK, 1), lambda t: (t, 0)),
        ],
        out_shape=[out1, out1, outf, outf],
        interpret=interpret,
    )(x, wg_pad, bg_pad)


def _sc_dispatch(x, p0, p1):
    per_w = N // SC_NW
    mesh = plsc.VectorSubcoreMesh(core_axis_name="c", subcore_axis_name="s")

    @functools.partial(
        pl.kernel, mesh=mesh, name="sc_dispatch_scatter",
        out_type=jax.ShapeDtypeStruct((S, D), jnp.float32),
        scratch_types=[
            pltpu.VMEM((per_w,), jnp.int32),
            pltpu.VMEM((per_w,), jnp.int32),
            pltpu.VMEM((per_w, D), jnp.float32),
            pltpu.SemaphoreType.DMA,
            pltpu.SemaphoreType.DMA,
        ],
    )
    def dispatch_k(x_hbm, p0_hbm, p1_hbm, out_hbm, ia, ib, xb, sa, sb):
        wid = lax.axis_index("s") * SC_NC + lax.axis_index("c")
        base = wid * per_w
        pltpu.sync_copy(p0_hbm.at[pl.ds(base, per_w)], ia)
        pltpu.sync_copy(p1_hbm.at[pl.ds(base, per_w)], ib)
        pltpu.sync_copy(x_hbm.at[pl.ds(base, per_w)], xb)
        c0 = pltpu.async_copy(xb, out_hbm.at[ia], sa)
        c1 = pltpu.async_copy(xb, out_hbm.at[ib], sb)
        c0.wait()
        c1.wait()

    return dispatch_k(x, p0, p1)


def _ffn_body(meta_ref, x_ref, we_ref, be_ref, hs_ref):
    t = pl.program_id(0)
    n_used = meta_ref[T_PAD]

    @pl.when(t < n_used)
    def _():
        h = jnp.dot(x_ref[...], we_ref[0],
                    preferred_element_type=jnp.float32)
        h = jnp.exp(jnp.maximum(h + be_ref[0], 0.0))
        hs_ref[...] = h * (1.0 / jnp.sum(h, axis=1, keepdims=True))


def _ffn(meta, x_pad, we, be, *, interpret=False):
    grid_spec = pltpu.PrefetchScalarGridSpec(
        num_scalar_prefetch=1,
        grid=(T_PAD,),
        in_specs=[
            pl.BlockSpec((BT, D), lambda t, m: (t, 0)),
            pl.BlockSpec((1, D, D), lambda t, m: (m[t], 0, 0)),
            pl.BlockSpec((1, 1, D), lambda t, m: (m[t], 0, 0)),
        ],
        out_specs=pl.BlockSpec((BT, D), lambda t, m: (t, 0)),
    )
    return pl.pallas_call(
        _ffn_body,
        grid_spec=grid_spec,
        out_shape=jax.ShapeDtypeStruct((S, D), jnp.float32),
        interpret=interpret,
    )(meta, x_pad, we, be)


def _sc_combine(hs, pos0, pos1, g0, g1):
    per_w = N // SC_NW
    ch = 32
    iters = per_w // ch
    mesh = plsc.VectorSubcoreMesh(core_axis_name="c", subcore_axis_name="s")

    @functools.partial(
        pl.kernel, mesh=mesh, name="sc_combine",
        compiler_params=_sc_compiler_params(),
        out_type=jax.ShapeDtypeStruct((N, D), jnp.float32),
        scratch_types=[
            pltpu.VMEM((ch,), jnp.int32),
            pltpu.VMEM((ch,), jnp.int32),
            pltpu.VMEM((ch,), jnp.float32),
            pltpu.VMEM((ch,), jnp.float32),
            pltpu.VMEM((ch, D), jnp.float32),
            pltpu.VMEM((ch, D), jnp.float32),
            pltpu.SemaphoreType.DMA,
            pltpu.SemaphoreType.DMA,
        ],
    )
    def combine_k(hs_hbm, p0_hbm, p1_hbm, g0_hbm, g1_hbm, out_hbm,
                  i0_v, i1_v, g0_v, g1_v, r0_v, r1_v, sem0, sem1):
        wid = lax.axis_index("s") * SC_NC + lax.axis_index("c")
        base0 = wid * per_w
        for k in range(iters):
            base = base0 + k * ch
            pltpu.sync_copy(p0_hbm.at[pl.ds(base, ch)], i0_v)
            pltpu.sync_copy(p1_hbm.at[pl.ds(base, ch)], i1_v)
            pltpu.sync_copy(g0_hbm.at[pl.ds(base, ch)], g0_v)
            pltpu.sync_copy(g1_hbm.at[pl.ds(base, ch)], g1_v)
            c0 = pltpu.async_copy(hs_hbm.at[i0_v], r0_v, sem0)
            c1 = pltpu.async_copy(hs_hbm.at[i1_v], r1_v, sem1)
            c0.wait()
            c1.wait()

            @pl.loop(0, ch)
            def _(r):
                lane = jnp.zeros((16,), jnp.int32) + r
                ga = plsc.load_gather(g0_v, [lane])
                gb = plsc.load_gather(g1_v, [lane])

                @pl.loop(0, D, step=128)
                def _(c):
                    for u in range(8):
                        slc = (r, pl.ds(c + u * 16, 16))
                        r0_v.at[slc][...] = (r0_v.at[slc][...] * ga
                                             + r1_v.at[slc][...] * gb)

            pltpu.sync_copy(r0_v, out_hbm.at[pl.ds(base, ch)])

    return combine_k(hs, pos0, pos1, g0, g1)


def _dispatch_plan(top_i, gate):
    flat_e = top_i.reshape(NTOP)
    onehot = (flat_e[:, None] == jnp.arange(E, dtype=jnp.int32)[None, :])
    onehot = onehot.astype(jnp.int32)
    counts = jnp.sum(onehot, axis=0)
    rank = jnp.sum((jnp.cumsum(onehot, axis=0) - onehot) * onehot, axis=1)
    psize = ((counts + BT - 1) // BT) * BT
    pstart = jnp.concatenate([jnp.zeros((1,), jnp.int32),
                              jnp.cumsum(psize)]).astype(jnp.int32)
    pos_flat = pstart[flat_e] + rank
    n_used = pstart[E] // BT
    t_idx = jnp.arange(T_PAD, dtype=jnp.int32)
    raw = jnp.searchsorted(pstart[1:], t_idx * BT, side="right").astype(jnp.int32)
    raw = jnp.minimum(raw, E - 1)
    last_e = raw[jnp.maximum(n_used - 1, 0)]
    e_of_tile = jnp.where(t_idx < n_used, raw, last_e)
    meta = jnp.concatenate([e_of_tile, n_used[None]]).astype(jnp.int32)
    pos0 = pos_flat[0::2]
    pos1 = pos_flat[1::2]
    return meta, pos0, pos1


def kernel(inputs, Wg, bg, We, be):
    wg_pad = jnp.zeros((D, EPAD), jnp.float32).at[:, :E].set(Wg)
    bg_pad = jnp.zeros((1, EPAD), jnp.float32).at[0, :E].set(bg)
    i1, i2, q1, q2 = _router(inputs, wg_pad, bg_pad)
    top_i = jnp.concatenate([i1, i2], axis=1)
    gate = jnp.concatenate([q1, q2], axis=1)
    meta, pos0, pos1 = _dispatch_plan(top_i, gate)
    x_pad = _sc_dispatch(inputs, pos0, pos1)
    hs = _ffn(meta, x_pad, We, be[:, None, :])
    return _sc_combine(hs, pos0, pos1, gate[:, 0], gate[:, 1])


# speedup vs baseline: 1.2392x; 1.2392x over previous
import dataclasses
import functools

import jax
import jax.numpy as jnp
from jax import lax
from jax.experimental import pallas as pl
from jax.experimental.pallas import tpu as pltpu
from jax.experimental.pallas import tpu_sc as plsc

E = 8
TOPX = 2
D = 1024
N = 2048
NTOP = N * TOPX

BT = 512
S = NTOP + E * BT
T_PAD = S // BT

BTOK = 512
EPAD = 128

SC_NC = 2
SC_NS = 16
SC_NW = SC_NC * SC_NS

NEG = -1e30


def _sc_compiler_params():
    cp = pltpu.CompilerParams()
    if "needs_layout_passes" in pltpu.CompilerParams.__dataclass_fields__:
        cp = dataclasses.replace(cp, needs_layout_passes=False)
    return cp


def _router_body(x_ref, wg_ref, bg_ref, i1_ref, i2_ref, q1_ref, q2_ref):
    logits = jnp.dot(x_ref[...], wg_ref[...],
                     preferred_element_type=jnp.float32)
    logits = logits + bg_ref[...]
    ids = lax.broadcasted_iota(jnp.int32, logits.shape, 1)
    valid = ids < E
    logits = jnp.where(valid, logits, NEG)
    m1 = jnp.max(logits, axis=1, keepdims=True)
    i1 = jnp.min(jnp.where(logits == m1, ids, E), axis=1, keepdims=True)
    rest = jnp.where((ids != i1) & valid, logits, NEG)
    m2 = jnp.max(rest, axis=1, keepdims=True)
    i2 = jnp.min(jnp.where(rest == m2, ids, E), axis=1, keepdims=True)
    ed = jnp.exp(m2 - m1)
    p1 = 1.0 / (1.0 + ed)
    p2 = ed / (1.0 + ed)
    e21 = jnp.exp(p2 - p1)
    i1_ref[...] = i1
    i2_ref[...] = i2
    q1_ref[...] = 1.0 / (1.0 + e21)
    q2_ref[...] = e21 / (1.0 + e21)


def _router(x, wg_pad, bg_pad, *, interpret=False):
    grid = (N // BTOK,)
    out1 = jax.ShapeDtypeStruct((N, 1), jnp.int32)
    outf = jax.ShapeDtypeStruct((N, 1), jnp.float32)
    return pl.pallas_call(
        _router_body,
        grid=grid,
        in_specs=[
            pl.BlockSpec((BTOK, D), lambda t: (t, 0)),
            pl.BlockSpec((D, EPAD), lambda t: (0, 0)),
            pl.BlockSpec((1, EPAD), lambda t: (0, 0)),
        ],
        out_specs=[
            pl.BlockSpec((BTOK, 1), lambda t: (t, 0)),
            pl.BlockSpec((BTOK, 1), lambda t: (t, 0)),
            pl.BlockSpec((BTOK, 1), lambda t: (t, 0)),
            pl.BlockSpec((BTOK, 1), lambda t: (t, 0)),
        ],
        out_shape=[out1, out1, outf, outf],
        interpret=interpret,
    )(x, wg_pad, bg_pad)


def _sc_dispatch(x, p0, p1):
    per_w = N // SC_NW
    mesh = plsc.VectorSubcoreMesh(core_axis_name="c", subcore_axis_name="s")

    @functools.partial(
        pl.kernel, mesh=mesh, name="sc_dispatch_scatter",
        out_type=jax.ShapeDtypeStruct((S, D), jnp.float32),
        scratch_types=[
            pltpu.VMEM((per_w,), jnp.int32),
            pltpu.VMEM((per_w,), jnp.int32),
            pltpu.VMEM((per_w, D), jnp.float32),
            pltpu.SemaphoreType.DMA,
            pltpu.SemaphoreType.DMA,
        ],
    )
    def dispatch_k(x_hbm, p0_hbm, p1_hbm, out_hbm, ia, ib, xb, sa, sb):
        wid = lax.axis_index("s") * SC_NC + lax.axis_index("c")
        base = wid * per_w
        pltpu.sync_copy(p0_hbm.at[pl.ds(base, per_w)], ia)
        pltpu.sync_copy(p1_hbm.at[pl.ds(base, per_w)], ib)
        pltpu.sync_copy(x_hbm.at[pl.ds(base, per_w)], xb)
        c0 = pltpu.async_copy(xb, out_hbm.at[ia], sa)
        c1 = pltpu.async_copy(xb, out_hbm.at[ib], sb)
        c0.wait()
        c1.wait()

    return dispatch_k(x, p0, p1)


def _ffn_body(meta_ref, x_ref, we_ref, be_ref, hs_ref):
    t = pl.program_id(0)
    n_used = meta_ref[T_PAD]

    @pl.when(t < n_used)
    def _():
        h = jnp.dot(x_ref[...], we_ref[0],
                    preferred_element_type=jnp.float32)
        h = jnp.exp(jnp.maximum(h + be_ref[0], 0.0))
        hs_ref[...] = h * (1.0 / jnp.sum(h, axis=1, keepdims=True))


def _ffn(meta, x_pad, we, be, *, interpret=False):
    grid_spec = pltpu.PrefetchScalarGridSpec(
        num_scalar_prefetch=1,
        grid=(T_PAD,),
        in_specs=[
            pl.BlockSpec((BT, D), lambda t, m: (t, 0)),
            pl.BlockSpec((1, D, D), lambda t, m: (m[t], 0, 0)),
            pl.BlockSpec((1, 1, D), lambda t, m: (m[t], 0, 0)),
        ],
        out_specs=pl.BlockSpec((BT, D), lambda t, m: (t, 0)),
    )
    return pl.pallas_call(
        _ffn_body,
        grid_spec=grid_spec,
        out_shape=jax.ShapeDtypeStruct((S, D), jnp.float32),
        interpret=interpret,
    )(meta, x_pad, we, be)


def _sc_combine(hs, pos0, pos1, g0, g1):
    per_w = N // SC_NW
    ch = 32
    iters = per_w // ch
    mesh = plsc.VectorSubcoreMesh(core_axis_name="c", subcore_axis_name="s")

    @functools.partial(
        pl.kernel, mesh=mesh, name="sc_combine",
        compiler_params=_sc_compiler_params(),
        out_type=jax.ShapeDtypeStruct((N, D), jnp.float32),
        scratch_types=[
            pltpu.VMEM((ch,), jnp.int32),
            pltpu.VMEM((ch,), jnp.int32),
            pltpu.VMEM((ch,), jnp.float32),
            pltpu.VMEM((ch,), jnp.float32),
            pltpu.VMEM((ch, D), jnp.float32),
            pltpu.VMEM((ch, D), jnp.float32),
            pltpu.SemaphoreType.DMA,
            pltpu.SemaphoreType.DMA,
        ],
    )
    def combine_k(hs_hbm, p0_hbm, p1_hbm, g0_hbm, g1_hbm, out_hbm,
                  i0_v, i1_v, g0_v, g1_v, r0_v, r1_v, sem0, sem1):
        wid = lax.axis_index("s") * SC_NC + lax.axis_index("c")
        base0 = wid * per_w
        for k in range(iters):
            base = base0 + k * ch
            pltpu.sync_copy(p0_hbm.at[pl.ds(base, ch)], i0_v)
            pltpu.sync_copy(p1_hbm.at[pl.ds(base, ch)], i1_v)
            pltpu.sync_copy(g0_hbm.at[pl.ds(base, ch)], g0_v)
            pltpu.sync_copy(g1_hbm.at[pl.ds(base, ch)], g1_v)
            c0 = pltpu.async_copy(hs_hbm.at[i0_v], r0_v, sem0)
            c1 = pltpu.async_copy(hs_hbm.at[i1_v], r1_v, sem1)
            c0.wait()
            c1.wait()

            @pl.loop(0, ch)
            def _(r):
                lane = jnp.zeros((16,), jnp.int32) + r
                ga = plsc.load_gather(g0_v, [lane])
                gb = plsc.load_gather(g1_v, [lane])

                @pl.loop(0, D, step=128)
                def _(c):
                    for u in range(8):
                        slc = (r, pl.ds(c + u * 16, 16))
                        r0_v.at[slc][...] = (r0_v.at[slc][...] * ga
                                             + r1_v.at[slc][...] * gb)

            pltpu.sync_copy(r0_v, out_hbm.at[pl.ds(base, ch)])

    return combine_k(hs, pos0, pos1, g0, g1)


def _dispatch_plan(top_i, gate):
    flat_e = top_i.reshape(NTOP)
    onehot = (flat_e[:, None] == jnp.arange(E, dtype=jnp.int32)[None, :])
    onehot = onehot.astype(jnp.int32)
    counts = jnp.sum(onehot, axis=0)
    rank = jnp.sum((jnp.cumsum(onehot, axis=0) - onehot) * onehot, axis=1)
    psize = ((counts + BT - 1) // BT) * BT
    pstart = jnp.concatenate([jnp.zeros((1,), jnp.int32),
                              jnp.cumsum(psize)]).astype(jnp.int32)
    pos_flat = pstart[flat_e] + rank
    n_used = pstart[E] // BT
    t_idx = jnp.arange(T_PAD, dtype=jnp.int32)
    raw = jnp.searchsorted(pstart[1:], t_idx * BT, side="right").astype(jnp.int32)
    raw = jnp.minimum(raw, E - 1)
    last_e = raw[jnp.maximum(n_used - 1, 0)]
    e_of_tile = jnp.where(t_idx < n_used, raw, last_e)
    meta = jnp.concatenate([e_of_tile, n_used[None]]).astype(jnp.int32)
    pos0 = pos_flat[0::2]
    pos1 = pos_flat[1::2]
    return meta, pos0, pos1


def kernel(inputs, Wg, bg, We, be):
    wg_pad = jnp.zeros((D, EPAD), jnp.float32).at[:, :E].set(Wg)
    bg_pad = jnp.zeros((1, EPAD), jnp.float32).at[0, :E].set(bg)
    i1, i2, q1, q2 = _router(inputs, wg_pad, bg_pad)
    top_i = jnp.concatenate([i1, i2], axis=1)
    gate = jnp.concatenate([q1, q2], axis=1)
    meta, pos0, pos1 = _dispatch_plan(top_i, gate)
    x_pad = _sc_dispatch(inputs, pos0, pos1)
    hs = _ffn(meta, x_pad, We, be[:, None, :])
    return _sc_combine(hs, pos0, pos1, gate[:, 0], gate[:, 1])
